# Initial kernel scaffold; baseline (speedup 1.0000x reference)
import jax
import jax.numpy as jnp
from jax.experimental import pallas as pl


def kernel(x, edge_index, W1, b1, Wmu, bmu, Wvar, bvar):
    raise NotImplementedError("write your pallas kernel here")


# speedup vs baseline: 26.4755x; 26.4755x over previous
import functools

import jax
import jax.numpy as jnp
from jax import lax
from jax.experimental import pallas as pl
from jax.experimental.pallas import tpu as pltpu
from jax.experimental.pallas import tpu_sc as plsc

N = 10000
NP = 10240
E = 320000
NWK = 32
EPT = E // NWK
K = 80
NWIN = EPT // K
RPT = NP // 16
D = 64

_mesh = plsc.VectorSubcoreMesh(core_axis_name="c", subcore_axis_name="s")
_sc_params = pltpu.CompilerParams(use_tc_tiling_on_sc=False)


def _sc_deg_body(dst_hbm, deg_hbm, di_v, ones_v, zrow_v, acc_sh, sem):
    cid = lax.axis_index("c")
    sid = lax.axis_index("s")
    tid = cid * 16 + sid

    def zfill(i, _):
        zrow_v[pl.ds(i * 16, 16)] = jnp.zeros((16,), jnp.float32)
        return 0

    lax.fori_loop(0, RPT // 16, zfill, 0)
    for i in range(K // 16):
        ones_v[pl.ds(i * 16, 16)] = jnp.ones((16,), jnp.float32)
    pltpu.sync_copy(zrow_v, acc_sh.at[pl.ds(sid * RPT, RPT)])
    pltpu.async_copy(dst_hbm.at[tid], di_v, sem).wait()
    plsc.subcore_barrier()

    def body(w, _):
        pltpu.sync_copy(ones_v, acc_sh.at[di_v.at[w]], add=True)
        return 0

    lax.fori_loop(0, NWIN, body, 0)
    plsc.subcore_barrier()
    pltpu.sync_copy(acc_sh.at[pl.ds(sid * RPT, RPT)],
                    deg_hbm.at[cid, pl.ds(sid * RPT, RPT)])


_sc_deg = functools.partial(
    pl.kernel,
    out_type=jax.ShapeDtypeStruct((2, NP), jnp.float32),
    mesh=_mesh,
    scratch_types=[
        pltpu.VMEM((NWIN, K), jnp.int32),
        pltpu.VMEM((K,), jnp.float32),
        pltpu.VMEM((RPT,), jnp.float32),
        pltpu.VMEM_SHARED((NP,), jnp.float32),
        pltpu.SemaphoreType.DMA,
    ],
    compiler_params=_sc_params,
)(_sc_deg_body)


def _sc_prop_body(y_hbm, src_hbm, dst_hbm, out_hbm,
                  si_v, di_v, rows_v, zblk_v, acc_sh, sem):
    cid = lax.axis_index("c")
    sid = lax.axis_index("s")
    tid = cid * 16 + sid

    def zfill(i, _):
        zblk_v[i // 4, pl.ds((i % 4) * 16, 16)] = jnp.zeros((16,), jnp.float32)
        return 0

    lax.fori_loop(0, RPT * (D // 16), zfill, 0)
    pltpu.sync_copy(zblk_v, acc_sh.at[pl.ds(sid * RPT, RPT)])
    pltpu.async_copy(src_hbm.at[tid], si_v, sem).wait()
    pltpu.async_copy(dst_hbm.at[tid], di_v, sem).wait()
    plsc.subcore_barrier()

    def body(w, _):
        pltpu.async_copy(y_hbm.at[si_v.at[w]], rows_v, sem).wait()
        pltpu.sync_copy(rows_v, acc_sh.at[di_v.at[w]], add=True)
        return 0

    lax.fori_loop(0, NWIN, body, 0)
    plsc.subcore_barrier()
    pltpu.sync_copy(acc_sh.at[pl.ds(sid * RPT, RPT)],
                    out_hbm.at[cid, pl.ds(sid * RPT, RPT)])


_sc_prop = functools.partial(
    pl.kernel,
    out_type=jax.ShapeDtypeStruct((2, NP, D), jnp.float32),
    mesh=_mesh,
    scratch_types=[
        pltpu.VMEM((NWIN, K), jnp.int32),
        pltpu.VMEM((NWIN, K), jnp.int32),
        pltpu.VMEM((K, D), jnp.float32),
        pltpu.VMEM((RPT, D), jnp.float32),
        pltpu.VMEM_SHARED((NP, D), jnp.float32),
        pltpu.SemaphoreType.DMA,
    ],
    compiler_params=_sc_params,
)(_sc_prop_body)


_BLK = 1024


def _tc1_body(x_ref, w_ref, da_ref, db_ref, y1_ref, dinv_ref):
    deg = da_ref[0] + db_ref[0] + 1.0
    dinv = lax.rsqrt(deg)
    xw = jnp.dot(x_ref[...], w_ref[...], preferred_element_type=jnp.float32)
    y1_ref[...] = dinv * xw
    dinv_ref[...] = dinv


def _tc2_body(sa_ref, sb_ref, y1_ref, dinv_ref, b1_ref, wc_ref, y2_ref):
    dinv = dinv_ref[...]
    t = dinv * (sa_ref[0] + sb_ref[0] + y1_ref[...]) + b1_ref[...]
    h = jnp.maximum(t, 0.0)
    y2_ref[...] = dinv * jnp.dot(h, wc_ref[...], preferred_element_type=jnp.float32)


def _tc3_body(sa_ref, sb_ref, y2_ref, dinv_ref, bc_ref, out_ref):
    out_ref[...] = (dinv_ref[...] * (sa_ref[0] + sb_ref[0] + y2_ref[...])
                    + bc_ref[...])


def _row_spec(cols):
    return pl.BlockSpec((_BLK, cols), lambda i: (i, 0))


def _half_spec(half, cols):
    return pl.BlockSpec((1, _BLK, cols), lambda i, h=half: (h, i, 0))


def _full_spec(rows, cols):
    return pl.BlockSpec((rows, cols), lambda i: (0, 0))


_tc1 = pl.pallas_call(
    _tc1_body,
    grid=(NP // _BLK,),
    in_specs=[_row_spec(128), _full_spec(128, 64),
              _half_spec(0, 1), _half_spec(1, 1)],
    out_specs=[_row_spec(D), _row_spec(1)],
    out_shape=[jax.ShapeDtypeStruct((NP, D), jnp.float32),
               jax.ShapeDtypeStruct((NP, 1), jnp.float32)],
)

_tc2 = pl.pallas_call(
    _tc2_body,
    grid=(NP // _BLK,),
    in_specs=[_half_spec(0, D), _half_spec(1, D), _row_spec(D), _row_spec(1),
              _full_spec(1, D), _full_spec(D, D)],
    out_specs=_row_spec(D),
    out_shape=jax.ShapeDtypeStruct((NP, D), jnp.float32),
)

_tc3 = pl.pallas_call(
    _tc3_body,
    grid=(NP // _BLK,),
    in_specs=[_half_spec(0, D), _half_spec(1, D), _row_spec(D), _row_spec(1),
              _full_spec(1, D)],
    out_specs=_row_spec(D),
    out_shape=jax.ShapeDtypeStruct((NP, D), jnp.float32),
)


def kernel(x, edge_index, W1, b1, Wmu, bmu, Wvar, bvar):
    ei = edge_index.astype(jnp.int32)
    src3 = ei[0].reshape(NWK, NWIN, K)
    dst3 = ei[1].reshape(NWK, NWIN, K)
    xp = jnp.pad(x, ((0, NP - N), (0, 0)))

    deg = _sc_deg(dst3)
    y1, dinv = _tc1(xp, W1, deg.reshape(2, NP, 1), deg.reshape(2, NP, 1))

    s1 = _sc_prop(y1, src3, dst3)
    wcat = jnp.concatenate([Wmu, Wvar], axis=1)
    bcat = jnp.concatenate([bmu, bvar]).reshape(1, D)
    y2 = _tc2(s1, s1, y1, dinv, b1.reshape(1, D), wcat)

    s2 = _sc_prop(y2, src3, dst3)
    out = _tc3(s2, s2, y2, dinv, bcat)
    return out[:N, :32], out[:N, 32:]


# speedup vs baseline: 31.1810x; 1.1777x over previous
import functools

import jax
import jax.numpy as jnp
from jax import lax
from jax.experimental import pallas as pl
from jax.experimental.pallas import tpu as pltpu
from jax.experimental.pallas import tpu_sc as plsc

N = 10000
NP = 10240
E = 320000
NWK = 32
EPT = E // NWK
K = 80
NWIN = EPT // K
RPT = NP // 16
D = 64

_mesh = plsc.VectorSubcoreMesh(core_axis_name="c", subcore_axis_name="s")
_sc_params = pltpu.CompilerParams(use_tc_tiling_on_sc=False)


def _sc_deg_body(dst_hbm, deg_hbm, di_v, ones_v, zrow_v, acc_sh, sem):
    cid = lax.axis_index("c")
    sid = lax.axis_index("s")
    tid = cid * 16 + sid

    def zfill(i, _):
        zrow_v[pl.ds(i * 16, 16)] = jnp.zeros((16,), jnp.float32)
        return 0

    lax.fori_loop(0, RPT // 16, zfill, 0)
    for i in range(K // 16):
        ones_v[pl.ds(i * 16, 16)] = jnp.ones((16,), jnp.float32)
    pltpu.sync_copy(zrow_v, acc_sh.at[pl.ds(sid * RPT, RPT)])
    pltpu.async_copy(dst_hbm.at[tid], di_v, sem).wait()
    plsc.subcore_barrier()

    CH = 25

    def body(c, _):
        def fire(w, _):
            pltpu.async_copy(ones_v, acc_sh.at[di_v.at[c * CH + w]], sem,
                             add=True)
            return 0

        lax.fori_loop(0, CH, fire, 0)

        def drain(w, _):
            pltpu.make_async_copy(ones_v, acc_sh.at[di_v.at[c * CH + w]],
                                  sem).wait()
            return 0

        lax.fori_loop(0, CH, drain, 0)
        return 0

    lax.fori_loop(0, NWIN // CH, body, 0)
    plsc.subcore_barrier()
    pltpu.sync_copy(acc_sh.at[pl.ds(sid * RPT, RPT)],
                    deg_hbm.at[cid, pl.ds(sid * RPT, RPT)])


_sc_deg = functools.partial(
    pl.kernel,
    out_type=jax.ShapeDtypeStruct((2, NP), jnp.float32),
    mesh=_mesh,
    scratch_types=[
        pltpu.VMEM((NWIN, K), jnp.int32),
        pltpu.VMEM((K,), jnp.float32),
        pltpu.VMEM((RPT,), jnp.float32),
        pltpu.VMEM_SHARED((NP,), jnp.float32),
        pltpu.SemaphoreType.DMA,
    ],
    compiler_params=_sc_params,
)(_sc_deg_body)


def _sc_prop_body(y_hbm, src_hbm, dst_hbm, out_hbm,
                  si_v, di_v, rows_a, rows_b, zblk_v, acc_sh,
                  sem_ga, sem_gb, sem):
    cid = lax.axis_index("c")
    sid = lax.axis_index("s")
    tid = cid * 16 + sid

    def zfill(i, _):
        zblk_v[i // 4, pl.ds((i % 4) * 16, 16)] = jnp.zeros((16,), jnp.float32)
        return 0

    lax.fori_loop(0, RPT * (D // 16), zfill, 0)
    pltpu.sync_copy(zblk_v, acc_sh.at[pl.ds(sid * RPT, RPT)])
    pltpu.async_copy(src_hbm.at[tid], si_v, sem).wait()
    pltpu.async_copy(dst_hbm.at[tid], di_v, sem).wait()
    plsc.subcore_barrier()

    def gath(w, buf, gsem):
        pltpu.async_copy(y_hbm.at[si_v.at[w]], buf, gsem)

    def gwait(buf, gsem):
        pltpu.make_async_copy(y_hbm.at[si_v.at[0]], buf, gsem).wait()

    def scat(w, buf):
        pltpu.sync_copy(buf, acc_sh.at[di_v.at[w]], add=True)

    gath(0, rows_a, sem_ga)

    def body(i, _):
        w = 2 * i
        gwait(rows_a, sem_ga)
        gath(w + 1, rows_b, sem_gb)
        scat(w, rows_a)
        gwait(rows_b, sem_gb)
        gath(w + 2, rows_a, sem_ga)
        scat(w + 1, rows_b)
        return 0

    lax.fori_loop(0, (NWIN - 1) // 2, body, 0)
    gwait(rows_a, sem_ga)
    scat(NWIN - 1, rows_a)
    plsc.subcore_barrier()
    pltpu.sync_copy(acc_sh.at[pl.ds(sid * RPT, RPT)],
                    out_hbm.at[cid, pl.ds(sid * RPT, RPT)])


_sc_prop = functools.partial(
    pl.kernel,
    out_type=jax.ShapeDtypeStruct((2, NP, D), jnp.float32),
    mesh=_mesh,
    scratch_types=[
        pltpu.VMEM((NWIN, K), jnp.int32),
        pltpu.VMEM((NWIN, K), jnp.int32),
        pltpu.VMEM((K, D), jnp.float32),
        pltpu.VMEM((K, D), jnp.float32),
        pltpu.VMEM((RPT, D), jnp.float32),
        pltpu.VMEM_SHARED((NP, D), jnp.float32),
        pltpu.SemaphoreType.DMA,
        pltpu.SemaphoreType.DMA,
        pltpu.SemaphoreType.DMA,
    ],
    compiler_params=_sc_params,
)(_sc_prop_body)


_BLK = 1024


def _tc1_body(x_ref, w_ref, da_ref, db_ref, y1_ref, dinv_ref):
    deg = da_ref[0] + db_ref[0] + 1.0
    dinv = lax.rsqrt(deg)
    xw = jnp.dot(x_ref[...], w_ref[...], preferred_element_type=jnp.float32)
    y1_ref[...] = dinv * xw
    dinv_ref[...] = dinv


def _tc2_body(sa_ref, sb_ref, y1_ref, dinv_ref, b1_ref, wc_ref, y2_ref):
    dinv = dinv_ref[...]
    t = dinv * (sa_ref[0] + sb_ref[0] + y1_ref[...]) + b1_ref[...]
    h = jnp.maximum(t, 0.0)
    y2_ref[...] = dinv * jnp.dot(h, wc_ref[...], preferred_element_type=jnp.float32)


def _tc3_body(sa_ref, sb_ref, y2_ref, dinv_ref, bc_ref, out_ref):
    out_ref[...] = (dinv_ref[...] * (sa_ref[0] + sb_ref[0] + y2_ref[...])
                    + bc_ref[...])


def _row_spec(cols):
    return pl.BlockSpec((_BLK, cols), lambda i: (i, 0))


def _half_spec(half, cols):
    return pl.BlockSpec((1, _BLK, cols), lambda i, h=half: (h, i, 0))


def _full_spec(rows, cols):
    return pl.BlockSpec((rows, cols), lambda i: (0, 0))


_tc1 = pl.pallas_call(
    _tc1_body,
    grid=(NP // _BLK,),
    in_specs=[_row_spec(128), _full_spec(128, 64),
              _half_spec(0, 1), _half_spec(1, 1)],
    out_specs=[_row_spec(D), _row_spec(1)],
    out_shape=[jax.ShapeDtypeStruct((NP, D), jnp.float32),
               jax.ShapeDtypeStruct((NP, 1), jnp.float32)],
)

_tc2 = pl.pallas_call(
    _tc2_body,
    grid=(NP // _BLK,),
    in_specs=[_half_spec(0, D), _half_spec(1, D), _row_spec(D), _row_spec(1),
              _full_spec(1, D), _full_spec(D, D)],
    out_specs=_row_spec(D),
    out_shape=jax.ShapeDtypeStruct((NP, D), jnp.float32),
)

_tc3 = pl.pallas_call(
    _tc3_body,
    grid=(NP // _BLK,),
    in_specs=[_half_spec(0, D), _half_spec(1, D), _row_spec(D), _row_spec(1),
              _full_spec(1, D)],
    out_specs=_row_spec(D),
    out_shape=jax.ShapeDtypeStruct((NP, D), jnp.float32),
)


def kernel(x, edge_index, W1, b1, Wmu, bmu, Wvar, bvar):
    ei = edge_index.astype(jnp.int32)
    src3 = ei[0].reshape(NWK, NWIN, K)
    dst3 = ei[1].reshape(NWK, NWIN, K)
    xp = jnp.pad(x, ((0, NP - N), (0, 0)))

    deg = _sc_deg(dst3)
    y1, dinv = _tc1(xp, W1, deg.reshape(2, NP, 1), deg.reshape(2, NP, 1))

    s1 = _sc_prop(y1, src3, dst3)
    wcat = jnp.concatenate([Wmu, Wvar], axis=1)
    bcat = jnp.concatenate([bmu, bvar]).reshape(1, D)
    y2 = _tc2(s1, s1, y1, dinv, b1.reshape(1, D), wcat)

    s2 = _sc_prop(y2, src3, dst3)
    out = _tc3(s2, s2, y2, dinv, bcat)
    return out[:N, :32], out[:N, 32:]


# speedup vs baseline: 40.1600x; 1.2880x over previous
import functools

import jax
import jax.numpy as jnp
from jax import lax
from jax.experimental import pallas as pl
from jax.experimental.pallas import tpu as pltpu
from jax.experimental.pallas import tpu_sc as plsc

N = 10000
NP = 10240
E = 320000
NWK = 32
EPT = E // NWK
K = 80
NWIN = EPT // K
RPT = NP // 16
D = 64

_mesh = plsc.VectorSubcoreMesh(core_axis_name="c", subcore_axis_name="s")
_sc_params = pltpu.CompilerParams(use_tc_tiling_on_sc=False)


def _sc_deg_body(dst_hbm, deg_hbm, di_v, ones_v, zrow_v, acc_sh, sem):
    cid = lax.axis_index("c")
    sid = lax.axis_index("s")
    tid = cid * 16 + sid

    def zfill(i, _):
        zrow_v[pl.ds(i * 16, 16)] = jnp.zeros((16,), jnp.float32)
        return 0

    lax.fori_loop(0, RPT // 16, zfill, 0)
    for i in range(K // 16):
        ones_v[pl.ds(i * 16, 16)] = jnp.ones((16,), jnp.float32)
    pltpu.sync_copy(zrow_v, acc_sh.at[pl.ds(sid * RPT, RPT)])
    pltpu.async_copy(dst_hbm.at[tid], di_v, sem).wait()
    plsc.subcore_barrier()

    CH = 25

    def body(c, _):
        def fire(w, _):
            pltpu.async_copy(ones_v, acc_sh.at[di_v.at[c * CH + w]], sem,
                             add=True)
            return 0

        lax.fori_loop(0, CH, fire, 0)

        def drain(w, _):
            pltpu.make_async_copy(ones_v, acc_sh.at[di_v.at[c * CH + w]],
                                  sem).wait()
            return 0

        lax.fori_loop(0, CH, drain, 0)
        return 0

    lax.fori_loop(0, NWIN // CH, body, 0)
    plsc.subcore_barrier()
    pltpu.sync_copy(acc_sh.at[pl.ds(sid * RPT, RPT)],
                    deg_hbm.at[cid, pl.ds(sid * RPT, RPT)])


_sc_deg = functools.partial(
    pl.kernel,
    out_type=jax.ShapeDtypeStruct((2, NP), jnp.float32),
    mesh=_mesh,
    scratch_types=[
        pltpu.VMEM((NWIN, K), jnp.int32),
        pltpu.VMEM((K,), jnp.float32),
        pltpu.VMEM((RPT,), jnp.float32),
        pltpu.VMEM_SHARED((NP,), jnp.float32),
        pltpu.SemaphoreType.DMA,
    ],
    compiler_params=_sc_params,
)(_sc_deg_body)


_NB = 5


def _sc_prop_body(y_hbm, src_hbm, dst_hbm, out_hbm,
                  si_v, di_v, rows0, rows1, rows2, rows3, rows4, zblk_v,
                  acc_sh, g0, g1, g2, g3, g4, s0, s1, s2, s3, s4, sem):
    cid = lax.axis_index("c")
    sid = lax.axis_index("s")
    tid = cid * 16 + sid
    rows = (rows0, rows1, rows2, rows3, rows4)
    gs = (g0, g1, g2, g3, g4)
    ss = (s0, s1, s2, s3, s4)

    def zfill(i, _):
        zblk_v[i // 4, pl.ds((i % 4) * 16, 16)] = jnp.zeros((16,), jnp.float32)
        return 0

    lax.fori_loop(0, RPT * (D // 16), zfill, 0)
    pltpu.sync_copy(zblk_v, acc_sh.at[pl.ds(sid * RPT, RPT)])
    pltpu.async_copy(src_hbm.at[tid], si_v, sem).wait()
    pltpu.async_copy(dst_hbm.at[tid], di_v, sem).wait()
    plsc.subcore_barrier()

    def gath(w, b):
        pltpu.async_copy(y_hbm.at[si_v.at[w]], rows[b], gs[b])

    def gwait(b):
        pltpu.make_async_copy(y_hbm.at[si_v.at[0]], rows[b], gs[b]).wait()

    def sstart(w, b):
        pltpu.async_copy(rows[b], acc_sh.at[di_v.at[w]], ss[b], add=True)

    def swait(b):
        pltpu.make_async_copy(rows[b], acc_sh.at[di_v.at[0]], ss[b]).wait()

    gath(0, 0)
    gath(1, 1)
    for w in (0, 1, 2):
        gwait(w % _NB)
        sstart(w, w % _NB)
        gath(w + 2, (w + 2) % _NB)

    def main(c, _):
        for j in range(_NB):
            b = (3 + j) % _NB
            w = 3 + _NB * c + j
            gwait(b)
            sstart(w, b)
            b2 = (b + 2) % _NB
            swait(b2)
            gath(w + 2, b2)
        return 0

    lax.fori_loop(0, (NWIN - _NB) // _NB, main, 0)
    for w in (NWIN - 2, NWIN - 1):
        b = w % _NB
        gwait(b)
        sstart(w, b)
        swait((b + 2) % _NB)
    for w in (NWIN - 3, NWIN - 2, NWIN - 1):
        swait(w % _NB)
    plsc.subcore_barrier()
    pltpu.sync_copy(acc_sh.at[pl.ds(sid * RPT, RPT)],
                    out_hbm.at[cid, pl.ds(sid * RPT, RPT)])


_sc_prop = functools.partial(
    pl.kernel,
    out_type=jax.ShapeDtypeStruct((2, NP, D), jnp.float32),
    mesh=_mesh,
    scratch_types=[
        pltpu.VMEM((NWIN, K), jnp.int32),
        pltpu.VMEM((NWIN, K), jnp.int32),
    ] + [pltpu.VMEM((K, D), jnp.float32) for _ in range(_NB)] + [
        pltpu.VMEM((RPT, D), jnp.float32),
        pltpu.VMEM_SHARED((NP, D), jnp.float32),
    ] + [pltpu.SemaphoreType.DMA for _ in range(2 * _NB + 1)],
    compiler_params=_sc_params,
)(_sc_prop_body)


_BLK = 1024


def _tc1_body(x_ref, w_ref, da_ref, db_ref, y1_ref, dinv_ref):
    deg = da_ref[0] + db_ref[0] + 1.0
    dinv = lax.rsqrt(deg)
    xw = jnp.dot(x_ref[...], w_ref[...], preferred_element_type=jnp.float32)
    y1_ref[...] = dinv * xw
    dinv_ref[...] = dinv


def _tc2_body(sa_ref, sb_ref, y1_ref, dinv_ref, b1_ref, wc_ref, y2_ref):
    dinv = dinv_ref[...]
    t = dinv * (sa_ref[0] + sb_ref[0] + y1_ref[...]) + b1_ref[...]
    h = jnp.maximum(t, 0.0)
    y2_ref[...] = dinv * jnp.dot(h, wc_ref[...], preferred_element_type=jnp.float32)


def _tc3_body(sa_ref, sb_ref, y2_ref, dinv_ref, bc_ref, out_ref):
    out_ref[...] = (dinv_ref[...] * (sa_ref[0] + sb_ref[0] + y2_ref[...])
                    + bc_ref[...])


def _row_spec(cols):
    return pl.BlockSpec((_BLK, cols), lambda i: (i, 0))


def _half_spec(half, cols):
    return pl.BlockSpec((1, _BLK, cols), lambda i, h=half: (h, i, 0))


def _full_spec(rows, cols):
    return pl.BlockSpec((rows, cols), lambda i: (0, 0))


_tc1 = pl.pallas_call(
    _tc1_body,
    grid=(NP // _BLK,),
    in_specs=[_row_spec(128), _full_spec(128, 64),
              _half_spec(0, 1), _half_spec(1, 1)],
    out_specs=[_row_spec(D), _row_spec(1)],
    out_shape=[jax.ShapeDtypeStruct((NP, D), jnp.float32),
               jax.ShapeDtypeStruct((NP, 1), jnp.float32)],
)

_tc2 = pl.pallas_call(
    _tc2_body,
    grid=(NP // _BLK,),
    in_specs=[_half_spec(0, D), _half_spec(1, D), _row_spec(D), _row_spec(1),
              _full_spec(1, D), _full_spec(D, D)],
    out_specs=_row_spec(D),
    out_shape=jax.ShapeDtypeStruct((NP, D), jnp.float32),
)

_tc3 = pl.pallas_call(
    _tc3_body,
    grid=(NP // _BLK,),
    in_specs=[_half_spec(0, D), _half_spec(1, D), _row_spec(D), _row_spec(1),
              _full_spec(1, D)],
    out_specs=_row_spec(D),
    out_shape=jax.ShapeDtypeStruct((NP, D), jnp.float32),
)


def kernel(x, edge_index, W1, b1, Wmu, bmu, Wvar, bvar):
    ei = edge_index.astype(jnp.int32)
    src3 = ei[0].reshape(NWK, NWIN, K)
    dst3 = ei[1].reshape(NWK, NWIN, K)
    xp = jnp.pad(x, ((0, NP - N), (0, 0)))

    deg = _sc_deg(dst3)
    y1, dinv = _tc1(xp, W1, deg.reshape(2, NP, 1), deg.reshape(2, NP, 1))

    s1 = _sc_prop(y1, src3, dst3)
    wcat = jnp.concatenate([Wmu, Wvar], axis=1)
    bcat = jnp.concatenate([bmu, bvar]).reshape(1, D)
    y2 = _tc2(s1, s1, y1, dinv, b1.reshape(1, D), wcat)

    s2 = _sc_prop(y2, src3, dst3)
    out = _tc3(s2, s2, y2, dinv, bcat)
    return out[:N, :32], out[:N, 32:]


# speedup vs baseline: 41.0398x; 1.0219x over previous
import functools

import jax
import jax.numpy as jnp
from jax import lax
from jax.experimental import pallas as pl
from jax.experimental.pallas import tpu as pltpu
from jax.experimental.pallas import tpu_sc as plsc

N = 10000
NP = 10240
E = 320000
NWK = 32
EPT = E // NWK
K = 80
NWIN = EPT // K
RPT = NP // 16
D = 64

_mesh = plsc.VectorSubcoreMesh(core_axis_name="c", subcore_axis_name="s")
_sc_params = pltpu.CompilerParams(use_tc_tiling_on_sc=False)


def _sc_deg_body(dst_hbm, deg_hbm, di_v, ones_v, zrow_v, acc_sh, sem):
    cid = lax.axis_index("c")
    sid = lax.axis_index("s")
    tid = cid * 16 + sid

    def zfill(i, _):
        zrow_v[pl.ds(i * 16, 16)] = jnp.zeros((16,), jnp.float32)
        return 0

    lax.fori_loop(0, RPT // 16, zfill, 0)
    for i in range(K // 16):
        ones_v[pl.ds(i * 16, 16)] = jnp.ones((16,), jnp.float32)
    pltpu.sync_copy(zrow_v, acc_sh.at[pl.ds(sid * RPT, RPT)])
    pltpu.async_copy(dst_hbm.at[tid], di_v, sem).wait()
    plsc.subcore_barrier()

    CH = 25

    def body(c, _):
        def fire(w, _):
            pltpu.async_copy(ones_v, acc_sh.at[di_v.at[c * CH + w]], sem,
                             add=True)
            return 0

        lax.fori_loop(0, CH, fire, 0)

        def drain(w, _):
            pltpu.make_async_copy(ones_v, acc_sh.at[di_v.at[c * CH + w]],
                                  sem).wait()
            return 0

        lax.fori_loop(0, CH, drain, 0)
        return 0

    lax.fori_loop(0, NWIN // CH, body, 0)
    plsc.subcore_barrier()
    pltpu.sync_copy(acc_sh.at[pl.ds(sid * RPT, RPT)],
                    deg_hbm.at[cid, pl.ds(sid * RPT, RPT)])


_sc_deg = functools.partial(
    pl.kernel,
    out_type=jax.ShapeDtypeStruct((2, NP), jnp.float32),
    mesh=_mesh,
    scratch_types=[
        pltpu.VMEM((NWIN, K), jnp.int32),
        pltpu.VMEM((K,), jnp.float32),
        pltpu.VMEM((RPT,), jnp.float32),
        pltpu.VMEM_SHARED((NP,), jnp.float32),
        pltpu.SemaphoreType.DMA,
    ],
    compiler_params=_sc_params,
)(_sc_deg_body)


_NB = 5


def _sc_prop_body(y_hbm, src_hbm, dst_hbm, out_hbm,
                  si_v, di_v, rows0, rows1, rows2, rows3, rows4, zblk_v,
                  acc_sh, g0, g1, g2, g3, g4, s0, s1, s2, s3, s4, sem):
    cid = lax.axis_index("c")
    sid = lax.axis_index("s")
    tid = cid * 16 + sid
    rows = (rows0, rows1, rows2, rows3, rows4)
    gs = (g0, g1, g2, g3, g4)
    ss = (s0, s1, s2, s3, s4)

    def zfill(i, _):
        zblk_v[i // 4, pl.ds((i % 4) * 16, 16)] = jnp.zeros((16,), jnp.float32)
        return 0

    lax.fori_loop(0, RPT * (D // 16), zfill, 0)
    pltpu.sync_copy(zblk_v, acc_sh.at[pl.ds(sid * RPT, RPT)])
    pltpu.async_copy(src_hbm.at[tid], si_v, sem).wait()
    pltpu.async_copy(dst_hbm.at[tid], di_v, sem).wait()
    plsc.subcore_barrier()

    def gath(w, b):
        pltpu.async_copy(y_hbm.at[si_v.at[w]], rows[b], gs[b])

    def gwait(b):
        pltpu.make_async_copy(y_hbm.at[si_v.at[0]], rows[b], gs[b]).wait()

    def sstart(w, b):
        pltpu.async_copy(rows[b], acc_sh.at[di_v.at[w]], ss[b], add=True)

    def swait(b):
        pltpu.make_async_copy(rows[b], acc_sh.at[di_v.at[0]], ss[b]).wait()

    gath(0, 0)
    gath(1, 1)
    for w in (0, 1, 2):
        gwait(w % _NB)
        sstart(w, w % _NB)
        gath(w + 2, (w + 2) % _NB)

    def main(c, _):
        for j in range(_NB):
            b = (3 + j) % _NB
            w = 3 + _NB * c + j
            gwait(b)
            sstart(w, b)
            b2 = (b + 2) % _NB
            swait(b2)
            gath(w + 2, b2)
        return 0

    lax.fori_loop(0, (NWIN - _NB) // _NB, main, 0)
    for w in (NWIN - 2, NWIN - 1):
        b = w % _NB
        gwait(b)
        sstart(w, b)
        swait((b + 2) % _NB)
    for w in (NWIN - 3, NWIN - 2, NWIN - 1):
        swait(w % _NB)
    plsc.subcore_barrier()
    pltpu.sync_copy(acc_sh.at[pl.ds(sid * RPT, RPT)],
                    out_hbm.at[cid, pl.ds(sid * RPT, RPT)])


_sc_prop = functools.partial(
    pl.kernel,
    out_type=jax.ShapeDtypeStruct((2, NP, D), jnp.float32),
    mesh=_mesh,
    scratch_types=[
        pltpu.VMEM((NWIN, K), jnp.int32),
        pltpu.VMEM((NWIN, K), jnp.int32),
    ] + [pltpu.VMEM((K, D), jnp.float32) for _ in range(_NB)] + [
        pltpu.VMEM((RPT, D), jnp.float32),
        pltpu.VMEM_SHARED((NP, D), jnp.float32),
    ] + [pltpu.SemaphoreType.DMA for _ in range(2 * _NB + 1)],
    compiler_params=_sc_params,
)(_sc_prop_body)


_BLK = 1000


def _tc1_body(x_ref, w_ref, da_ref, db_ref, y1_ref, dinv_ref):
    deg = da_ref[0] + db_ref[0] + 1.0
    dinv = lax.rsqrt(deg)
    xw = jnp.dot(x_ref[...], w_ref[...], preferred_element_type=jnp.float32)
    y1_ref[...] = dinv * xw
    dinv_ref[...] = dinv


def _tc2_body(sa_ref, sb_ref, y1_ref, dinv_ref, b1_ref, wc_ref, y2_ref):
    dinv = dinv_ref[...]
    t = dinv * (sa_ref[0] + sb_ref[0] + y1_ref[...]) + b1_ref[...]
    h = jnp.maximum(t, 0.0)
    y2_ref[...] = dinv * jnp.dot(h, wc_ref[...], preferred_element_type=jnp.float32)


def _tc3_body(sa_ref, sb_ref, y2_ref, dinv_ref, bmu_ref, bvar_ref,
              mu_ref, lv_ref):
    s = dinv_ref[...] * (sa_ref[0] + sb_ref[0] + y2_ref[...])
    mu_ref[...] = s[:, :32] + bmu_ref[...]
    lv_ref[...] = s[:, 32:] + bvar_ref[...]


def _row_spec(cols):
    return pl.BlockSpec((_BLK, cols), lambda i: (i, 0))


def _half_spec(half, cols):
    return pl.BlockSpec((1, _BLK, cols), lambda i, h=half: (h, i, 0))


def _full_spec(rows, cols):
    return pl.BlockSpec((rows, cols), lambda i: (0, 0))


_tc1 = pl.pallas_call(
    _tc1_body,
    grid=(N // _BLK,),
    in_specs=[_row_spec(128), _full_spec(128, 64),
              _half_spec(0, 1), _half_spec(1, 1)],
    out_specs=[_row_spec(D), _row_spec(1)],
    out_shape=[jax.ShapeDtypeStruct((N, D), jnp.float32),
               jax.ShapeDtypeStruct((N, 1), jnp.float32)],
)

_tc2 = pl.pallas_call(
    _tc2_body,
    grid=(N // _BLK,),
    in_specs=[_half_spec(0, D), _half_spec(1, D), _row_spec(D), _row_spec(1),
              _full_spec(1, D), _full_spec(D, D)],
    out_specs=_row_spec(D),
    out_shape=jax.ShapeDtypeStruct((N, D), jnp.float32),
)

_tc3 = pl.pallas_call(
    _tc3_body,
    grid=(N // _BLK,),
    in_specs=[_half_spec(0, D), _half_spec(1, D), _row_spec(D), _row_spec(1),
              _full_spec(1, 32), _full_spec(1, 32)],
    out_specs=[_row_spec(32), _row_spec(32)],
    out_shape=[jax.ShapeDtypeStruct((N, 32), jnp.float32),
               jax.ShapeDtypeStruct((N, 32), jnp.float32)],
)


def kernel(x, edge_index, W1, b1, Wmu, bmu, Wvar, bvar):
    ei = edge_index.astype(jnp.int32)
    src3 = ei[0].reshape(NWK, NWIN, K)
    dst3 = ei[1].reshape(NWK, NWIN, K)

    deg = _sc_deg(dst3)
    deg3 = deg.reshape(2, NP, 1)
    y1, dinv = _tc1(x, W1, deg3, deg3)

    s1 = _sc_prop(y1, src3, dst3)
    wcat = jnp.concatenate([Wmu, Wvar], axis=1)
    y2 = _tc2(s1, s1, y1, dinv, b1.reshape(1, D), wcat)

    s2 = _sc_prop(y2, src3, dst3)
    mu, logvar = _tc3(s2, s2, y2, dinv,
                      bmu.reshape(1, 32), bvar.reshape(1, 32))
    return mu, logvar


# speedup vs baseline: 43.2215x; 1.0532x over previous
import functools

import jax
import jax.numpy as jnp
from jax import lax
from jax.experimental import pallas as pl
from jax.experimental.pallas import tpu as pltpu
from jax.experimental.pallas import tpu_sc as plsc

N = 10000
NP = 10240
E = 320000
NWK = 32
EPT = E // NWK
K = 80
NWIN = EPT // K
RPT = NP // 16
D = 64

_mesh = plsc.VectorSubcoreMesh(core_axis_name="c", subcore_axis_name="s")
_sc_params = pltpu.CompilerParams(use_tc_tiling_on_sc=False)


def _sc_deg_body(dst_hbm, deg_hbm, di_v, ones_v, zrow_v, acc_sh, sem):
    cid = lax.axis_index("c")
    sid = lax.axis_index("s")
    tid = cid * 16 + sid

    def zfill(i, _):
        zrow_v[pl.ds(i * 16, 16)] = jnp.zeros((16,), jnp.float32)
        return 0

    lax.fori_loop(0, RPT // 16, zfill, 0)
    for i in range(K // 16):
        ones_v[pl.ds(i * 16, 16)] = jnp.ones((16,), jnp.float32)
    pltpu.sync_copy(zrow_v, acc_sh.at[pl.ds(sid * RPT, RPT)])
    pltpu.async_copy(dst_hbm.at[tid], di_v, sem).wait()
    plsc.subcore_barrier()

    CH = 25

    def body(c, _):
        def fire(w, _):
            pltpu.async_copy(ones_v, acc_sh.at[di_v.at[c * CH + w]], sem,
                             add=True)
            return 0

        lax.fori_loop(0, CH, fire, 0)

        def drain(w, _):
            pltpu.make_async_copy(ones_v, acc_sh.at[di_v.at[c * CH + w]],
                                  sem).wait()
            return 0

        lax.fori_loop(0, CH, drain, 0)
        return 0

    lax.fori_loop(0, NWIN // CH, body, 0)
    plsc.subcore_barrier()
    pltpu.sync_copy(acc_sh.at[pl.ds(sid * RPT, RPT)],
                    deg_hbm.at[cid, pl.ds(sid * RPT, RPT)])


_sc_deg = functools.partial(
    pl.kernel,
    out_type=jax.ShapeDtypeStruct((2, NP), jnp.float32),
    mesh=_mesh,
    scratch_types=[
        pltpu.VMEM((NWIN, K), jnp.int32),
        pltpu.VMEM((K,), jnp.float32),
        pltpu.VMEM((RPT,), jnp.float32),
        pltpu.VMEM_SHARED((NP,), jnp.float32),
        pltpu.SemaphoreType.DMA,
    ],
    compiler_params=_sc_params,
)(_sc_deg_body)


_NB = 5


def _sc_prop_body(y_hbm, src_hbm, dst_hbm, out_hbm,
                  si_v, di_v, rows0, rows1, rows2, rows3, rows4, zblk_v,
                  acc_sh, g0, g1, g2, g3, g4, s0, s1, s2, s3, s4, sem):
    cid = lax.axis_index("c")
    sid = lax.axis_index("s")
    tid = cid * 16 + sid
    rows = (rows0, rows1, rows2, rows3, rows4)
    gs = (g0, g1, g2, g3, g4)
    ss = (s0, s1, s2, s3, s4)

    def zfill(i, _):
        zblk_v[i // 4, pl.ds((i % 4) * 16, 16)] = jnp.zeros((16,), jnp.float32)
        return 0

    lax.fori_loop(0, RPT * (D // 16), zfill, 0)
    pltpu.sync_copy(zblk_v, acc_sh.at[pl.ds(sid * RPT, RPT)])
    pltpu.async_copy(src_hbm.at[tid], si_v, sem).wait()
    pltpu.async_copy(dst_hbm.at[tid], di_v, sem).wait()
    plsc.subcore_barrier()

    def gath(w, b):
        pltpu.async_copy(y_hbm.at[si_v.at[w]], rows[b], gs[b])

    def gwait(b):
        pltpu.make_async_copy(y_hbm.at[si_v.at[0]], rows[b], gs[b]).wait()

    def sstart(w, b):
        pltpu.async_copy(rows[b], acc_sh.at[di_v.at[w]], ss[b], add=True)

    def swait(b):
        pltpu.make_async_copy(rows[b], acc_sh.at[di_v.at[0]], ss[b]).wait()

    gath(0, 0)
    gath(1, 1)
    for w in (0, 1, 2):
        gwait(w % _NB)
        sstart(w, w % _NB)
        gath(w + 2, (w + 2) % _NB)

    def main(c, _):
        for j in range(_NB):
            b = (3 + j) % _NB
            w = 3 + _NB * c + j
            gwait(b)
            sstart(w, b)
            b2 = (b + 2) % _NB
            swait(b2)
            gath(w + 2, b2)
        return 0

    lax.fori_loop(0, (NWIN - _NB) // _NB, main, 0)
    for w in (NWIN - 2, NWIN - 1):
        b = w % _NB
        gwait(b)
        sstart(w, b)
        swait((b + 2) % _NB)
    for w in (NWIN - 3, NWIN - 2, NWIN - 1):
        swait(w % _NB)
    plsc.subcore_barrier()
    pltpu.sync_copy(acc_sh.at[pl.ds(sid * RPT, RPT)],
                    out_hbm.at[cid, pl.ds(sid * RPT, RPT)])


_sc_prop = functools.partial(
    pl.kernel,
    out_type=jax.ShapeDtypeStruct((2, NP, D), jnp.float32),
    mesh=_mesh,
    scratch_types=[
        pltpu.VMEM((NWIN, K), jnp.int32),
        pltpu.VMEM((NWIN, K), jnp.int32),
    ] + [pltpu.VMEM((K, D), jnp.float32) for _ in range(_NB)] + [
        pltpu.VMEM((RPT, D), jnp.float32),
        pltpu.VMEM_SHARED((NP, D), jnp.float32),
    ] + [pltpu.SemaphoreType.DMA for _ in range(2 * _NB + 1)],
    compiler_params=_sc_params,
)(_sc_prop_body)


_BLK = 1024


def _tc1_body(x_ref, w_ref, dg_ref, y1_ref, dinv_ref):
    deg = dg_ref[0:1, :] + dg_ref[1:2, :] + 1.0
    dinv = jnp.reshape(lax.rsqrt(deg), (_BLK, 1))
    xw = jnp.dot(x_ref[...], w_ref[...], preferred_element_type=jnp.float32)
    y1_ref[...] = dinv * xw
    dinv_ref[...] = dinv


def _tc2_body(sa_ref, sb_ref, y1_ref, dinv_ref, b1_ref, wc_ref, y2_ref):
    dinv = dinv_ref[...]
    t = dinv * (sa_ref[0] + sb_ref[0] + y1_ref[...]) + b1_ref[...]
    h = jnp.maximum(t, 0.0)
    y2_ref[...] = dinv * jnp.dot(h, wc_ref[...], preferred_element_type=jnp.float32)


def _tc3_body(sa_ref, sb_ref, y2_ref, dinv_ref, bmu_ref, bvar_ref,
              mu_ref, lv_ref):
    s = dinv_ref[...] * (sa_ref[0] + sb_ref[0] + y2_ref[...])
    mu_ref[...] = s[:, :32] + bmu_ref[...]
    lv_ref[...] = s[:, 32:] + bvar_ref[...]


def _row_spec(cols):
    return pl.BlockSpec((_BLK, cols), lambda i: (i, 0))


def _half_spec(half, cols):
    return pl.BlockSpec((1, _BLK, cols), lambda i, h=half: (h, i, 0))


def _deg_spec():
    return pl.BlockSpec((2, _BLK), lambda i: (0, i))


def _full_spec(rows, cols):
    return pl.BlockSpec((rows, cols), lambda i: (0, 0))


_tc1 = pl.pallas_call(
    _tc1_body,
    grid=(N // _BLK,),
    in_specs=[_row_spec(128), _full_spec(128, 64), _deg_spec()],
    out_specs=[_row_spec(D), _row_spec(1)],
    out_shape=[jax.ShapeDtypeStruct((N, D), jnp.float32),
               jax.ShapeDtypeStruct((N, 1), jnp.float32)],
)

_tc2 = pl.pallas_call(
    _tc2_body,
    grid=(N // _BLK,),
    in_specs=[_half_spec(0, D), _half_spec(1, D), _row_spec(D), _row_spec(1),
              _full_spec(1, D), _full_spec(D, D)],
    out_specs=_row_spec(D),
    out_shape=jax.ShapeDtypeStruct((N, D), jnp.float32),
)

_tc3 = pl.pallas_call(
    _tc3_body,
    grid=(N // _BLK,),
    in_specs=[_half_spec(0, D), _half_spec(1, D), _row_spec(D), _row_spec(1),
              _full_spec(1, 32), _full_spec(1, 32)],
    out_specs=[_row_spec(32), _row_spec(32)],
    out_shape=[jax.ShapeDtypeStruct((N, 32), jnp.float32),
               jax.ShapeDtypeStruct((N, 32), jnp.float32)],
)


def kernel(x, edge_index, W1, b1, Wmu, bmu, Wvar, bvar):
    ei = edge_index.astype(jnp.int32)
    src3 = ei[0].reshape(NWK, NWIN, K)
    dst3 = ei[1].reshape(NWK, NWIN, K)

    deg = _sc_deg(dst3)
    y1, dinv = _tc1(x, W1, deg)

    s1 = _sc_prop(y1, src3, dst3)
    wcat = jnp.concatenate([Wmu, Wvar], axis=1)
    y2 = _tc2(s1, s1, y1, dinv, b1.reshape(1, D), wcat)

    s2 = _sc_prop(y2, src3, dst3)
    mu, logvar = _tc3(s2, s2, y2, dinv,
                      bmu.reshape(1, 32), bvar.reshape(1, 32))
    return mu, logvar
